# baseline (speedup 1.0000x reference)
import functools

import jax
import jax.numpy as jnp
from jax import lax
from jax.experimental import pallas as pl
from jax.experimental.pallas import tpu as pltpu
from jax.experimental.pallas import tpu_sc as plsc

_NUM_AGENTS = 2
_N_STATES = 1_000_000
_NUM_ACTIONS = 16
_BATCH = 4096
_NDIM = 3
_STRIDE0 = 10_000
_STRIDE1 = 100

_NC = 2
_NS = 16
_NW = _NC * _NS
_BPW = _BATCH // _NW
_L = 16


def _sc_gather(x_flat, table):
    mesh = plsc.VectorSubcoreMesh(core_axis_name="c", subcore_axis_name="s")

    @functools.partial(
        pl.kernel,
        mesh=mesh,
        compiler_params=pltpu.CompilerParams(use_tc_tiling_on_sc=False),
        out_type=jax.ShapeDtypeStruct((_NUM_AGENTS * _BATCH, _NUM_ACTIONS),
                                      jnp.float32),
        scratch_types=[
            pltpu.VMEM((_BPW,), jnp.int32),
            pltpu.VMEM((_BPW,), jnp.int32),
            pltpu.VMEM((_BPW,), jnp.int32),
            pltpu.VMEM((_BPW,), jnp.int32),
            pltpu.VMEM((_BPW,), jnp.int32),
            pltpu.VMEM((_BPW, _NUM_ACTIONS), jnp.float32),
            pltpu.VMEM((_BPW, _NUM_ACTIONS), jnp.float32),
            pltpu.SemaphoreType.DMA,
        ],
    )
    def k(x_hbm, table_hbm, out_hbm,
          x0_v, x1_v, x2_v, idx0_v, idx1_v, rows0_v, rows1_v, sem):
        wid = lax.axis_index("s") * _NC + lax.axis_index("c")
        base = wid * _BPW
        pltpu.sync_copy(x_hbm.at[pl.ds(base, _BPW)], x0_v)
        pltpu.sync_copy(x_hbm.at[pl.ds(_BATCH + base, _BPW)], x1_v)
        pltpu.sync_copy(x_hbm.at[pl.ds(2 * _BATCH + base, _BPW)], x2_v)
        for j in range(_BPW // _L):
            s = pl.ds(j * _L, _L)
            idx = x0_v[s] * _STRIDE0 + x1_v[s] * _STRIDE1 + x2_v[s]
            idx0_v[s] = idx
            idx1_v[s] = idx + _N_STATES
        g0 = pltpu.async_copy(table_hbm.at[idx0_v], rows0_v, sem)
        g1 = pltpu.async_copy(table_hbm.at[idx1_v], rows1_v, sem)
        g0.wait()
        pltpu.sync_copy(rows0_v, out_hbm.at[pl.ds(base, _BPW)])
        g1.wait()
        pltpu.sync_copy(rows1_v, out_hbm.at[pl.ds(_BATCH + base, _BPW)])

    return k(x_flat, table)


def kernel(x, params):
    x_flat = x.reshape(_NDIM * _BATCH)
    table = params.reshape(_NUM_AGENTS * _N_STATES, _NUM_ACTIONS)
    out = _sc_gather(x_flat, table)
    return out.reshape(_NUM_AGENTS, _BATCH, _NUM_ACTIONS)
